# baseline (speedup 1.0000x reference)
import functools

import jax
import jax.numpy as jnp
from jax import lax
from jax.experimental import pallas as pl
from jax.experimental.pallas import tpu as pltpu
from jax.experimental.pallas import tpu_sc as plsc

_N = 10000
_E = 320000
_DIN = 128
_H = 256
_NG = 64
_NOUT = 24

_NC = 2
_NS = 16
_NW = _NC * _NS
_HALF = _N // _NC
_OWN = 312
_OWN_LAST = _HALF - (_NS - 1) * _OWN
_SH_ROWS = 5120
_TRASH = 5119
_DCOL = 128

_EB = 2048
_NB = 160
_EPAD = _EB * _NB
_K = 128
_Q = 4
_CAP_ROWS = 2576


def _prep(srcf, dstf):
    mesh = plsc.VectorSubcoreMesh(core_axis_name="c", subcore_axis_name="s")

    @functools.partial(
        pl.kernel,
        mesh=mesh,
        compiler_params=pltpu.CompilerParams(needs_layout_passes=False),
        out_type=(
            jax.ShapeDtypeStruct((_NW, _CAP_ROWS, _K), jnp.int32),
            jax.ShapeDtypeStruct((_NW, _CAP_ROWS, _K), jnp.int32),
            jax.ShapeDtypeStruct((_NW, 16), jnp.int32),
        ),
        scratch_types=[
            pltpu.VMEM((_EB,), jnp.int32),
            pltpu.VMEM((_EB,), jnp.int32),
            pltpu.VMEM((32, _K), jnp.int32),
            pltpu.VMEM((32, _K), jnp.int32),
            pltpu.VMEM((16,), jnp.int32),
        ],
    )
    def k(src_hbm, dst_hbm, cs_hbm, cd_hbm, counts_hbm,
          sblk, dblk, rs, rd, cnt_ref):
        c = lax.axis_index("c")
        s = lax.axis_index("s")
        w = c * _NS + s
        scbase = c * _HALF
        lo = scbase + s * _OWN
        n_own = jnp.where(s == _NS - 1, _OWN_LAST, _OWN)
        hi = lo + n_own

        cnt_ref[...] = jnp.zeros((16,), jnp.int32)

        def blk_body(b, flushed):
            pltpu.sync_copy(src_hbm.at[pl.ds(b * _EB, _EB)], sblk)
            pltpu.sync_copy(dst_hbm.at[pl.ds(b * _EB, _EB)], dblk)

            def cbody(i, carry):
                cnt = cnt_ref[...]
                sv = sblk[pl.ds(i * 16, 16)]
                dv = dblk[pl.ds(i * 16, 16)]
                m = (dv >= jnp.full((16,), lo, jnp.int32)) & \
                    (dv < jnp.full((16,), hi, jnp.int32))
                pos = cnt + plsc.cumsum(m.astype(jnp.int32)) - 1
                prow = (pos // _K) % 32
                pcol = pos % _K
                plsc.store_scatter(rs, [prow, pcol], sv, mask=m)
                plsc.store_scatter(
                    rd, [prow, pcol],
                    dv - jnp.full((16,), scbase, jnp.int32), mask=m)
                cnt_ref[...] = cnt + plsc.all_reduce_population_count(m)
                return carry

            lax.fori_loop(0, _EB // 16, cbody, 0)

            f = cnt_ref[...][0]
            cond = (f - flushed * _K) >= 2048
            fl16 = pl.multiple_of(flushed, 16)
            h = (flushed // 16) % 2

            @pl.when(cond)
            def _():
                pltpu.sync_copy(rs.at[pl.ds(h * 16, 16)],
                                cs_hbm.at[w, pl.ds(fl16, 16)])
                pltpu.sync_copy(rd.at[pl.ds(h * 16, 16)],
                                cd_hbm.at[w, pl.ds(fl16, 16)])

            return jnp.where(cond, flushed + 16, flushed)

        flushed = lax.fori_loop(0, _NB, blk_body, jnp.int32(0))

        cnt = cnt_ref[...]
        lanes = lax.iota(jnp.int32, 16)
        for t in range(512 // 16):
            pos = cnt + lanes + 16 * t
            plsc.store_scatter(rs, [(pos // _K) % 32, pos % _K],
                               jnp.zeros((16,), jnp.int32))
            plsc.store_scatter(rd, [(pos // _K) % 32, pos % _K],
                               jnp.full((16,), _TRASH, jnp.int32))
        f_pad = cnt[0] + 512
        for _ in range(2):
            cond = flushed * _K < f_pad
            fl16 = pl.multiple_of(flushed, 16)
            h = (flushed // 16) % 2

            @pl.when(cond)
            def _():
                pltpu.sync_copy(rs.at[pl.ds(h * 16, 16)],
                                cs_hbm.at[w, pl.ds(fl16, 16)])
                pltpu.sync_copy(rd.at[pl.ds(h * 16, 16)],
                                cd_hbm.at[w, pl.ds(fl16, 16)])

            flushed = jnp.where(cond, flushed + 16, flushed)

        pltpu.sync_copy(cnt_ref, counts_hbm.at[w])

    return k(srcf, dstf)


def _aggregate(feats, cs, cd, counts, zrows):
    np_ = len(feats)
    mesh = plsc.VectorSubcoreMesh(core_axis_name="c", subcore_axis_name="s")

    @functools.partial(
        pl.kernel,
        mesh=mesh,
        compiler_params=pltpu.CompilerParams(needs_layout_passes=False),
        out_type=tuple(jax.ShapeDtypeStruct((_N, _DCOL), jnp.float32)
                       for _ in range(np_)),
        scratch_types=[
            pltpu.VMEM((16, _K), jnp.int32),
            pltpu.VMEM((16, _K), jnp.int32),
            pltpu.VMEM((16,), jnp.int32),
            pltpu.VMEM((_K, _DCOL), jnp.float32),
            pltpu.VMEM((_K, _DCOL), jnp.float32),
            pltpu.VMEM((_K, _DCOL), jnp.float32),
            pltpu.VMEM((_K, _DCOL), jnp.float32),
            pltpu.VMEM_SHARED((_SH_ROWS, _DCOL), jnp.float32),
            pltpu.SemaphoreType.DMA,
            pltpu.SemaphoreType.DMA,
            pltpu.SemaphoreType.DMA,
            pltpu.SemaphoreType.DMA,
        ],
    )
    def k(*refs):
        feat_hbm = refs[:np_]
        cs_hbm, cd_hbm, counts_hbm, zero_hbm = refs[np_:np_ + 4]
        out_hbm = refs[np_ + 4:2 * np_ + 4]
        (sbuf, dbuf, cbuf, r0, r1, r2, r3, acc,
         g0, g1, g2, g3) = refs[2 * np_ + 4:]
        rows = (r0, r1, r2, r3)
        sems = (g0, g1, g2, g3)

        c = lax.axis_index("c")
        s = lax.axis_index("s")
        w = c * _NS + s
        base = c * _HALF
        lo_local = s * _OWN

        pltpu.sync_copy(counts_hbm.at[w], cbuf)
        cnt = cbuf[...]
        nq = (cnt[0] + _Q * _K - 1) // (_Q * _K)
        nblk = (nq + 3) // 4

        for p in range(np_):
            @pl.when(s < _NS - 1)
            def _():
                pltpu.sync_copy(zero_hbm.at[pl.ds(0, _OWN)],
                                acc.at[pl.ds(lo_local, _OWN)])

            @pl.when(s == _NS - 1)
            def _():
                pltpu.sync_copy(zero_hbm,
                                acc.at[pl.ds(lo_local, _OWN_LAST)])

            def blk_body(b, carry):
                b16 = pl.multiple_of(b * 16, 16)
                pltpu.sync_copy(cs_hbm.at[w, pl.ds(b16, 16)], sbuf)
                pltpu.sync_copy(cd_hbm.at[w, pl.ds(b16, 16)], dbuf)
                q_here = jnp.minimum(4, nq - b * 4)

                def qbody(q, carry2):
                    for u in range(_Q):
                        pltpu.async_copy(
                            feat_hbm[p].at[sbuf.at[q * _Q + u]], rows[u],
                            sems[u]).wait()
                        pltpu.sync_copy(rows[u],
                                        acc.at[dbuf.at[q * _Q + u]], add=True)
                    return carry2

                lax.fori_loop(0, q_here, qbody, 0)
                return carry

            lax.fori_loop(0, nblk, blk_body, 0)

            @pl.when(s < _NS - 1)
            def _():
                pltpu.sync_copy(acc.at[pl.ds(lo_local, _OWN)],
                                out_hbm[p].at[pl.ds(base + lo_local, _OWN)])

            @pl.when(s == _NS - 1)
            def _():
                pltpu.sync_copy(
                    acc.at[pl.ds(lo_local, _OWN_LAST)],
                    out_hbm[p].at[pl.ds(base + lo_local, _OWN_LAST)])

    return k(*feats, cs, cd, counts, zrows)


def _layer_tc(aggr_halves, h, wr_halves, wro, bias):
    nh = len(aggr_halves)
    dh = h.shape[1]
    blk = 1000
    grid = _N // blk

    def body(*refs):
        a_refs = refs[:nh]
        h_ref = refs[nh]
        wr_refs = refs[nh + 1:2 * nh + 1]
        wro_ref, b_ref, o_ref = refs[2 * nh + 1:]
        acc = jnp.dot(a_refs[0][...], wr_refs[0][...],
                      preferred_element_type=jnp.float32)
        for i in range(1, nh):
            acc += jnp.dot(a_refs[i][...], wr_refs[i][...],
                           preferred_element_type=jnp.float32)
        acc += jnp.dot(h_ref[...], wro_ref[...],
                       preferred_element_type=jnp.float32)
        o_ref[...] = jnp.maximum(acc + b_ref[...], 0.0)

    in_specs = (
        [pl.BlockSpec((blk, _DCOL), lambda i: (i, 0)) for _ in range(nh)]
        + [pl.BlockSpec((blk, dh), lambda i: (i, 0))]
        + [pl.BlockSpec((_DCOL, _H), lambda i: (0, 0)) for _ in range(nh)]
        + [pl.BlockSpec((dh, _H), lambda i: (0, 0)),
           pl.BlockSpec((1, _H), lambda i: (0, 0))]
    )
    return pl.pallas_call(
        body,
        grid=(grid,),
        in_specs=in_specs,
        out_specs=pl.BlockSpec((blk, _H), lambda i: (i, 0)),
        out_shape=jax.ShapeDtypeStruct((_N, _H), jnp.float32),
    )(*aggr_halves, h, *wr_halves, wro, bias.reshape(1, _H))


def _pool_head(h, batch_col, wout_pad, bout_pad):
    def body(h_ref, b_ref, w_ref, bo_ref, o_ref):
        onehot = (b_ref[...] == lax.broadcasted_iota(jnp.int32, (_N, _NG), 1))
        onehot = onehot.astype(jnp.float32)
        sums = lax.dot_general(onehot, h_ref[...], (((0,), (0,)), ((), ())),
                               preferred_element_type=jnp.float32,
                               precision=lax.Precision.HIGHEST)
        counts = jnp.sum(onehot, axis=0)[:, None]
        pooled = sums / jnp.maximum(counts, 1.0)
        o_ref[...] = jnp.dot(pooled, w_ref[...],
                             preferred_element_type=jnp.float32,
                             precision=lax.Precision.HIGHEST) + bo_ref[...]

    return pl.pallas_call(
        body,
        out_shape=jax.ShapeDtypeStruct((_NG, 128), jnp.float32),
    )(h, batch_col, wout_pad, bout_pad)


def kernel(x, edge_index, batch, Wrel1, Wroot1, b1, Wrel, Wroot, b, Wout, bout):
    src = edge_index[0].astype(jnp.int32)
    dst = edge_index[1].astype(jnp.int32)
    pad = _EPAD - _E
    srcf = jnp.concatenate([src, jnp.zeros((pad,), jnp.int32)])
    dstf = jnp.concatenate([dst, jnp.full((pad,), 2 * _N, jnp.int32)])

    cs, cd, counts = _prep(srcf, dstf)
    zrows = jnp.zeros((_OWN_LAST, _DCOL), jnp.float32)

    (a,) = _aggregate((x,), cs, cd, counts, zrows)
    h = _layer_tc((a,), x, (Wrel1,), Wroot1, b1)
    for i in range(6):
        halves = _aggregate((h[:, :_DCOL], h[:, _DCOL:]), cs, cd, counts, zrows)
        h = _layer_tc(halves, h, (Wrel[i, :_DCOL], Wrel[i, _DCOL:]),
                      Wroot[i], b[i])

    batch_col = batch.astype(jnp.int32).reshape(_N, 1)
    wout_pad = jnp.zeros((_H, 128), jnp.float32).at[:, :_NOUT].set(Wout)
    bout_pad = jnp.zeros((1, 128), jnp.float32).at[0, :_NOUT].set(bout)
    out = _pool_head(h, batch_col, wout_pad, bout_pad)
    return out[:, :_NOUT]


# speedup vs baseline: 1.0690x; 1.0690x over previous
import functools

import jax
import jax.numpy as jnp
from jax import lax
from jax.experimental import pallas as pl
from jax.experimental.pallas import tpu as pltpu
from jax.experimental.pallas import tpu_sc as plsc

_N = 10000
_E = 320000
_DIN = 128
_H = 256
_NG = 64
_NOUT = 24

_NC = 2
_NS = 16
_NW = _NC * _NS
_HALF = _N // _NC
_OWN = 312
_OWN_LAST = _HALF - (_NS - 1) * _OWN
_SH_ROWS = 5120
_TRASH = 5119
_DCOL = 128

_EB = 2048
_NB = 160
_EPAD = _EB * _NB
_K = 128
_Q = 4
_CAP_ROWS = 2576


def _prep(srcf, dstf):
    mesh = plsc.VectorSubcoreMesh(core_axis_name="c", subcore_axis_name="s")

    @functools.partial(
        pl.kernel,
        mesh=mesh,
        compiler_params=pltpu.CompilerParams(needs_layout_passes=False),
        out_type=(
            jax.ShapeDtypeStruct((_NW, _CAP_ROWS, _K), jnp.int32),
            jax.ShapeDtypeStruct((_NW, _CAP_ROWS, _K), jnp.int32),
            jax.ShapeDtypeStruct((_NW, 16), jnp.int32),
        ),
        scratch_types=[
            pltpu.VMEM((_EB,), jnp.int32),
            pltpu.VMEM((_EB,), jnp.int32),
            pltpu.VMEM((32, _K), jnp.int32),
            pltpu.VMEM((32, _K), jnp.int32),
            pltpu.VMEM((16,), jnp.int32),
        ],
    )
    def k(src_hbm, dst_hbm, cs_hbm, cd_hbm, counts_hbm,
          sblk, dblk, rs, rd, cnt_ref):
        c = lax.axis_index("c")
        s = lax.axis_index("s")
        w = c * _NS + s
        scbase = c * _HALF
        lo = scbase + s * _OWN
        n_own = jnp.where(s == _NS - 1, _OWN_LAST, _OWN)
        hi = lo + n_own

        cnt_ref[...] = jnp.zeros((16,), jnp.int32)

        def blk_body(b, flushed):
            pltpu.sync_copy(src_hbm.at[pl.ds(b * _EB, _EB)], sblk)
            pltpu.sync_copy(dst_hbm.at[pl.ds(b * _EB, _EB)], dblk)

            def cbody(i, carry):
                cnt = cnt_ref[...]
                sv = sblk[pl.ds(i * 16, 16)]
                dv = dblk[pl.ds(i * 16, 16)]
                m = (dv >= jnp.full((16,), lo, jnp.int32)) & \
                    (dv < jnp.full((16,), hi, jnp.int32))
                pos = cnt + plsc.cumsum(m.astype(jnp.int32)) - 1
                prow = (pos // _K) % 32
                pcol = pos % _K
                plsc.store_scatter(rs, [prow, pcol], sv, mask=m)
                plsc.store_scatter(
                    rd, [prow, pcol],
                    dv - jnp.full((16,), scbase, jnp.int32), mask=m)
                cnt_ref[...] = cnt + plsc.all_reduce_population_count(m)
                return carry

            lax.fori_loop(0, _EB // 16, cbody, 0)

            f = cnt_ref[...][0]
            cond = (f - flushed * _K) >= 2048
            fl16 = pl.multiple_of(flushed, 16)
            h = (flushed // 16) % 2

            @pl.when(cond)
            def _():
                pltpu.sync_copy(rs.at[pl.ds(h * 16, 16)],
                                cs_hbm.at[w, pl.ds(fl16, 16)])
                pltpu.sync_copy(rd.at[pl.ds(h * 16, 16)],
                                cd_hbm.at[w, pl.ds(fl16, 16)])

            return jnp.where(cond, flushed + 16, flushed)

        flushed = lax.fori_loop(0, _NB, blk_body, jnp.int32(0))

        cnt = cnt_ref[...]
        lanes = lax.iota(jnp.int32, 16)
        for t in range(512 // 16):
            pos = cnt + lanes + 16 * t
            plsc.store_scatter(rs, [(pos // _K) % 32, pos % _K],
                               jnp.zeros((16,), jnp.int32))
            plsc.store_scatter(rd, [(pos // _K) % 32, pos % _K],
                               jnp.full((16,), _TRASH, jnp.int32))
        f_pad = cnt[0] + 512
        for _ in range(2):
            cond = flushed * _K < f_pad
            fl16 = pl.multiple_of(flushed, 16)
            h = (flushed // 16) % 2

            @pl.when(cond)
            def _():
                pltpu.sync_copy(rs.at[pl.ds(h * 16, 16)],
                                cs_hbm.at[w, pl.ds(fl16, 16)])
                pltpu.sync_copy(rd.at[pl.ds(h * 16, 16)],
                                cd_hbm.at[w, pl.ds(fl16, 16)])

            flushed = jnp.where(cond, flushed + 16, flushed)

        pltpu.sync_copy(cnt_ref, counts_hbm.at[w])

    return k(srcf, dstf)


def _aggregate(feats, cs, cd, counts, zrows):
    np_ = len(feats)
    mesh = plsc.VectorSubcoreMesh(core_axis_name="c", subcore_axis_name="s")

    @functools.partial(
        pl.kernel,
        mesh=mesh,
        compiler_params=pltpu.CompilerParams(needs_layout_passes=False),
        out_type=tuple(jax.ShapeDtypeStruct((_N, _DCOL), jnp.float32)
                       for _ in range(np_)),
        scratch_types=[
            pltpu.VMEM((16, _K), jnp.int32),
            pltpu.VMEM((16, _K), jnp.int32),
            pltpu.VMEM((16,), jnp.int32),
            pltpu.VMEM((_K, _DCOL), jnp.float32),
            pltpu.VMEM((_K, _DCOL), jnp.float32),
            pltpu.VMEM((_K, _DCOL), jnp.float32),
            pltpu.VMEM((_K, _DCOL), jnp.float32),
            pltpu.VMEM_SHARED((_SH_ROWS, _DCOL), jnp.float32),
            pltpu.SemaphoreType.DMA,
            pltpu.SemaphoreType.DMA,
            pltpu.SemaphoreType.DMA,
            pltpu.SemaphoreType.DMA,
        ],
    )
    def k(*refs):
        feat_hbm = refs[:np_]
        cs_hbm, cd_hbm, counts_hbm, zero_hbm = refs[np_:np_ + 4]
        out_hbm = refs[np_ + 4:2 * np_ + 4]
        (sbuf, dbuf, cbuf, r0, r1, r2, r3, acc,
         g0, g1, g2, g3) = refs[2 * np_ + 4:]
        rows = (r0, r1, r2, r3)
        sems = (g0, g1, g2, g3)

        c = lax.axis_index("c")
        s = lax.axis_index("s")
        w = c * _NS + s
        base = c * _HALF
        lo_local = s * _OWN

        pltpu.sync_copy(counts_hbm.at[w], cbuf)
        cnt = cbuf[...]
        nq = (cnt[0] + _Q * _K - 1) // (_Q * _K)
        nblk = (nq + 3) // 4

        for p in range(np_):
            @pl.when(s < _NS - 1)
            def _():
                pltpu.sync_copy(zero_hbm.at[pl.ds(0, _OWN)],
                                acc.at[pl.ds(lo_local, _OWN)])

            @pl.when(s == _NS - 1)
            def _():
                pltpu.sync_copy(zero_hbm,
                                acc.at[pl.ds(lo_local, _OWN_LAST)])

            def blk_body(b, carry):
                b16 = pl.multiple_of(b * 16, 16)
                pltpu.sync_copy(cs_hbm.at[w, pl.ds(b16, 16)], sbuf)
                pltpu.sync_copy(cd_hbm.at[w, pl.ds(b16, 16)], dbuf)
                q_here = jnp.minimum(4, nq - b * 4)

                def qbody(q, carry2):
                    dsc = [pltpu.async_copy(
                        feat_hbm[p].at[sbuf.at[q * _Q + u]], rows[u], sems[u])
                        for u in range(_Q)]
                    for u in range(_Q):
                        dsc[u].wait()
                        pltpu.sync_copy(rows[u],
                                        acc.at[dbuf.at[q * _Q + u]], add=True)
                    return carry2

                lax.fori_loop(0, q_here, qbody, 0)
                return carry

            lax.fori_loop(0, nblk, blk_body, 0)

            @pl.when(s < _NS - 1)
            def _():
                pltpu.sync_copy(acc.at[pl.ds(lo_local, _OWN)],
                                out_hbm[p].at[pl.ds(base + lo_local, _OWN)])

            @pl.when(s == _NS - 1)
            def _():
                pltpu.sync_copy(
                    acc.at[pl.ds(lo_local, _OWN_LAST)],
                    out_hbm[p].at[pl.ds(base + lo_local, _OWN_LAST)])

    return k(*feats, cs, cd, counts, zrows)


def _layer_tc(aggr_halves, h, wr_halves, wro, bias):
    nh = len(aggr_halves)
    dh = h.shape[1]
    blk = 1000
    grid = _N // blk

    def body(*refs):
        a_refs = refs[:nh]
        h_ref = refs[nh]
        wr_refs = refs[nh + 1:2 * nh + 1]
        wro_ref, b_ref, o_ref = refs[2 * nh + 1:]
        acc = jnp.dot(a_refs[0][...], wr_refs[0][...],
                      preferred_element_type=jnp.float32)
        for i in range(1, nh):
            acc += jnp.dot(a_refs[i][...], wr_refs[i][...],
                           preferred_element_type=jnp.float32)
        acc += jnp.dot(h_ref[...], wro_ref[...],
                       preferred_element_type=jnp.float32)
        o_ref[...] = jnp.maximum(acc + b_ref[...], 0.0)

    in_specs = (
        [pl.BlockSpec((blk, _DCOL), lambda i: (i, 0)) for _ in range(nh)]
        + [pl.BlockSpec((blk, dh), lambda i: (i, 0))]
        + [pl.BlockSpec((_DCOL, _H), lambda i: (0, 0)) for _ in range(nh)]
        + [pl.BlockSpec((dh, _H), lambda i: (0, 0)),
           pl.BlockSpec((1, _H), lambda i: (0, 0))]
    )
    return pl.pallas_call(
        body,
        grid=(grid,),
        in_specs=in_specs,
        out_specs=pl.BlockSpec((blk, _H), lambda i: (i, 0)),
        out_shape=jax.ShapeDtypeStruct((_N, _H), jnp.float32),
    )(*aggr_halves, h, *wr_halves, wro, bias.reshape(1, _H))


def _pool_head(h, batch_col, wout_pad, bout_pad):
    def body(h_ref, b_ref, w_ref, bo_ref, o_ref):
        onehot = (b_ref[...] == lax.broadcasted_iota(jnp.int32, (_N, _NG), 1))
        onehot = onehot.astype(jnp.float32)
        sums = lax.dot_general(onehot, h_ref[...], (((0,), (0,)), ((), ())),
                               preferred_element_type=jnp.float32,
                               precision=lax.Precision.HIGHEST)
        counts = jnp.sum(onehot, axis=0)[:, None]
        pooled = sums / jnp.maximum(counts, 1.0)
        o_ref[...] = jnp.dot(pooled, w_ref[...],
                             preferred_element_type=jnp.float32,
                             precision=lax.Precision.HIGHEST) + bo_ref[...]

    return pl.pallas_call(
        body,
        out_shape=jax.ShapeDtypeStruct((_NG, 128), jnp.float32),
    )(h, batch_col, wout_pad, bout_pad)


def kernel(x, edge_index, batch, Wrel1, Wroot1, b1, Wrel, Wroot, b, Wout, bout):
    src = edge_index[0].astype(jnp.int32)
    dst = edge_index[1].astype(jnp.int32)
    pad = _EPAD - _E
    srcf = jnp.concatenate([src, jnp.zeros((pad,), jnp.int32)])
    dstf = jnp.concatenate([dst, jnp.full((pad,), 2 * _N, jnp.int32)])

    cs, cd, counts = _prep(srcf, dstf)
    zrows = jnp.zeros((_OWN_LAST, _DCOL), jnp.float32)

    (a,) = _aggregate((x,), cs, cd, counts, zrows)
    h = _layer_tc((a,), x, (Wrel1,), Wroot1, b1)
    for i in range(6):
        halves = _aggregate((h[:, :_DCOL], h[:, _DCOL:]), cs, cd, counts, zrows)
        h = _layer_tc(halves, h, (Wrel[i, :_DCOL], Wrel[i, _DCOL:]),
                      Wroot[i], b[i])

    batch_col = batch.astype(jnp.int32).reshape(_N, 1)
    wout_pad = jnp.zeros((_H, 128), jnp.float32).at[:, :_NOUT].set(Wout)
    bout_pad = jnp.zeros((1, 128), jnp.float32).at[0, :_NOUT].set(bout)
    out = _pool_head(h, batch_col, wout_pad, bout_pad)
    return out[:, :_NOUT]
